# baseline (speedup 1.0000x reference)
import jax
import jax.numpy as jnp
from jax import lax
from jax.experimental import pallas as pl
from jax.experimental.pallas import tpu as pltpu
from jax.experimental.pallas import tpu_sc as plsc

NUM_CORES = 2
NUM_SUBCORES = 16
NUM_WORKERS = NUM_CORES * NUM_SUBCORES
LANES = 16
BATCH = 16384
RANK = 32
BPW = BATCH // NUM_WORKERS
IDX_CHUNK = 128
NCHUNK = BPW // IDX_CHUNK


def _gmf_body(user_h, item_h, eu_h, ei_h, wb_h, out_h,
              idxu_v, idxi_v, ru_v, ri_v, pt_v, out_v, wb_v, sem):
    wid = lax.axis_index("s") * NUM_CORES + lax.axis_index("c")
    base = wid * BPW

    pltpu.sync_copy(wb_h, wb_v)
    for c in range(NCHUNK):
        pltpu.sync_copy(user_h.at[pl.ds(base + c * IDX_CHUNK, IDX_CHUNK)],
                        idxu_v.at[c])
        pltpu.sync_copy(item_h.at[pl.ds(base + c * IDX_CHUNK, IDX_CHUNK)],
                        idxi_v.at[c])

    copies = []
    for c in range(NCHUNK):
        copies.append(pltpu.async_copy(
            eu_h.at[idxu_v.at[c]], ru_v.at[pl.ds(c * IDX_CHUNK, IDX_CHUNK)], sem))
        copies.append(pltpu.async_copy(
            ei_h.at[idxi_v.at[c]], ri_v.at[pl.ds(c * IDX_CHUNK, IDX_CHUNK)], sem))
    for cp in copies:
        cp.wait()

    w0 = wb_v[pl.ds(0, LANES)]
    w1 = wb_v[pl.ds(LANES, LANES)]
    bv = wb_v[pl.ds(2 * LANES, LANES)]
    lane_iota = lax.broadcasted_iota(jnp.int32, (LANES,), 0)

    UNROLL = 8

    def stage_a(ib, carry):
        for k in range(UNROLL):
            i = ib * UNROLL + k
            ua, ub = plsc.unpack(ru_v[i, pl.ds(0, RANK)],
                                 format=plsc.PackFormat.INTERLEAVED)
            ia, ib_ = plsc.unpack(ri_v[i, pl.ds(0, RANK)],
                                  format=plsc.PackFormat.INTERLEAVED)
            p = ua * ia * w0 + ub * ib_ * w1
            plsc.store_scatter(pt_v, [lane_iota * BPW + i], p)
        return carry

    lax.fori_loop(0, BPW // UNROLL, stage_a, 0)

    def stage_b(bi, carry):
        acc = bv
        for l in range(LANES):
            acc = acc + pt_v[pl.ds(l * BPW + bi * LANES, LANES)]
        out_v[pl.ds(bi * LANES, LANES)] = acc
        return carry

    lax.fori_loop(0, BPW // LANES, stage_b, 0)

    pltpu.sync_copy(out_v, out_h.at[pl.ds(base, BPW)])


def kernel(user, item, embed_user, embed_item, W, b):
    w = W.reshape(RANK)
    wb = jnp.concatenate(
        [w[0::2], w[1::2],
         jnp.full((LANES,), b.reshape(-1)[0], jnp.float32)]
    ).astype(jnp.float32)

    run = pl.kernel(
        _gmf_body,
        out_type=jax.ShapeDtypeStruct((BATCH,), jnp.float32),
        mesh=plsc.VectorSubcoreMesh(core_axis_name="c", subcore_axis_name="s"),
        compiler_params=pltpu.CompilerParams(
            needs_layout_passes=False, use_tc_tiling_on_sc=False),
        scratch_types=[
            pltpu.VMEM((NCHUNK, IDX_CHUNK), jnp.int32),
            pltpu.VMEM((NCHUNK, IDX_CHUNK), jnp.int32),
            pltpu.VMEM((BPW, RANK), jnp.bfloat16),
            pltpu.VMEM((BPW, RANK), jnp.bfloat16),
            pltpu.VMEM((LANES * BPW,), jnp.float32),
            pltpu.VMEM((BPW,), jnp.float32),
            pltpu.VMEM((RANK + LANES,), jnp.float32),
            pltpu.SemaphoreType.DMA,
        ],
    )
    return run(user.astype(jnp.int32), item.astype(jnp.int32),
               embed_user.astype(jnp.bfloat16), embed_item.astype(jnp.bfloat16),
               wb)


# speedup vs baseline: 1.0153x; 1.0153x over previous
import jax
import jax.numpy as jnp
from jax import lax
from jax.experimental import pallas as pl
from jax.experimental.pallas import tpu as pltpu
from jax.experimental.pallas import tpu_sc as plsc

NUM_CORES = 2
NUM_SUBCORES = 16
NUM_WORKERS = NUM_CORES * NUM_SUBCORES
LANES = 16
BATCH = 16384
RANK = 32
NBAND = RANK // 8
NITEM = 1000001
GROUP = 512
NGROups_FULL = NITEM // GROUP
TAIL_ROWS = NITEM - NGROups_FULL * GROUP
GW = 16384
BPW = BATCH // NUM_WORKERS
IDX_CHUNK = 128
NCHUNK = BPW // IDX_CHUNK


def _relayout_body(ei3_h, rows_h, inb_v, outb_v, tail_v, tout_v,
                   sem_i0, sem_i1, sem_o0, sem_o1):
    wid = lax.axis_index("s") * NUM_CORES + lax.axis_index("c")
    trips = jnp.where(wid == 0, 62, 61)
    start_g = 61 * wid + jnp.minimum(wid, 1)

    def in_src(g):
        row0 = (start_g + g) * GROUP
        return ei3_h.at[:, :, pl.ds(row0, GROUP)]

    pltpu.async_copy(in_src(0), inb_v.at[0], sem_i0)
    pltpu.async_copy(in_src(1), inb_v.at[1], sem_i1)
    pltpu.async_copy(rows_h.at[pl.ds(0, GW)], outb_v.at[0], sem_o0)
    pltpu.async_copy(rows_h.at[pl.ds(0, GW)], outb_v.at[1], sem_o1)

    ci = lax.broadcasted_iota(jnp.int32, (LANES,), 0)
    k_lo = ci // 8
    j_vec = ci % 8
    k_hi = k_lo + 2

    def drain_in(par):
        pltpu.make_async_copy(in_src(0), inb_v.at[par],
                              sem_i0 if par == 0 else sem_i1).wait()

    def drain_out(par):
        pltpu.make_async_copy(rows_h.at[pl.ds(0, GW)], outb_v.at[par],
                              sem_o0 if par == 0 else sem_o1).wait()

    UNROLL = 4

    def group_iter(g, carry):
        par = g % 2
        par_b = jnp.full((LANES,), par, jnp.int32)
        row0 = (start_g + g) * GROUP
        nxt = jnp.minimum(g + 2, trips - 1)

        @pl.when(par == 0)
        def _():
            drain_out(0)
            drain_in(0)

        @pl.when(par == 1)
        def _():
            drain_out(1)
            drain_in(1)

        def rows_iter(xb, c2):
            for u in range(UNROLL):
                x = xb * UNROLL + u
                xv = jnp.full((LANES,), x, jnp.int32)
                lo = plsc.load_gather(inb_v, [par_b, k_lo, j_vec, xv])
                hi = plsc.load_gather(inb_v, [par_b, k_hi, j_vec, xv])
                outb_v[par, pl.ds(x * RANK, LANES)] = lo
                outb_v[par, pl.ds(x * RANK + LANES, LANES)] = hi
            return c2

        lax.fori_loop(0, GROUP // UNROLL, rows_iter, 0)

        @pl.when(par == 0)
        def _():
            pltpu.async_copy(outb_v.at[0],
                             rows_h.at[pl.ds(row0 * RANK, GW)], sem_o0)
            pltpu.async_copy(in_src(nxt), inb_v.at[0], sem_i0)

        @pl.when(par == 1)
        def _():
            pltpu.async_copy(outb_v.at[1],
                             rows_h.at[pl.ds(row0 * RANK, GW)], sem_o1)
            pltpu.async_copy(in_src(nxt), inb_v.at[1], sem_i1)

        return carry

    lax.fori_loop(0, trips, group_iter, 0)

    drain_in(0)
    drain_in(1)
    drain_out(0)
    drain_out(1)

    @pl.when(wid == NUM_WORKERS - 1)
    def _():
        t0 = NGROups_FULL * GROUP
        pltpu.sync_copy(ei3_h.at[:, :, pl.ds(t0, TAIL_ROWS)], tail_v)
        for x in range(TAIL_ROWS):
            xv = jnp.full((LANES,), x, jnp.int32)
            lo = plsc.load_gather(tail_v, [k_lo, j_vec, xv])
            hi = plsc.load_gather(tail_v, [k_hi, j_vec, xv])
            tout_v[pl.ds(x * RANK, LANES)] = lo
            tout_v[pl.ds(x * RANK + LANES, LANES)] = hi
        pltpu.sync_copy(tout_v, rows_h.at[pl.ds(t0 * RANK, TAIL_ROWS * RANK)])


def _gmf_body(user_h, item_h, eu_h, ei_h, wb_h, out_h,
              idxu_v, idxi_v, ru_v, ri_v, pt_v, out_v, wb_v, sem):
    wid = lax.axis_index("s") * NUM_CORES + lax.axis_index("c")
    base = wid * BPW

    pltpu.sync_copy(wb_h, wb_v)
    for c in range(NCHUNK):
        pltpu.sync_copy(user_h.at[pl.ds(base + c * IDX_CHUNK, IDX_CHUNK)],
                        idxu_v.at[c])
        pltpu.sync_copy(item_h.at[pl.ds(base + c * IDX_CHUNK, IDX_CHUNK)],
                        idxi_v.at[c])

    copies = []
    for c in range(NCHUNK):
        copies.append(pltpu.async_copy(
            eu_h.at[idxu_v.at[c]], ru_v.at[pl.ds(c * IDX_CHUNK, IDX_CHUNK)], sem))
        copies.append(pltpu.async_copy(
            ei_h.at[idxi_v.at[c]], ri_v.at[pl.ds(c * IDX_CHUNK, IDX_CHUNK)], sem))
    for cp in copies:
        cp.wait()

    w0 = wb_v[pl.ds(0, LANES)]
    w1 = wb_v[pl.ds(LANES, LANES)]
    bv = wb_v[pl.ds(2 * LANES, LANES)]
    lane_iota = lax.broadcasted_iota(jnp.int32, (LANES,), 0)

    UNROLL = 8

    def stage_a(ib, carry):
        for k in range(UNROLL):
            i = ib * UNROLL + k
            p = (ru_v[i, pl.ds(0, LANES)] * ri_v[i, pl.ds(0, LANES)] * w0
                 + ru_v[i, pl.ds(LANES, LANES)] * ri_v[i, pl.ds(LANES, LANES)] * w1)
            plsc.store_scatter(pt_v, [lane_iota * BPW + i], p)
        return carry

    lax.fori_loop(0, BPW // UNROLL, stage_a, 0)

    def stage_b(bi, carry):
        acc = bv
        for l in range(LANES):
            acc = acc + pt_v[pl.ds(l * BPW + bi * LANES, LANES)]
        out_v[pl.ds(bi * LANES, LANES)] = acc
        return carry

    lax.fori_loop(0, BPW // LANES, stage_b, 0)

    pltpu.sync_copy(out_v, out_h.at[pl.ds(base, BPW)])


def kernel(user, item, embed_user, embed_item, W, b):
    mesh = plsc.VectorSubcoreMesh(core_axis_name="c", subcore_axis_name="s")

    ei3 = embed_item.T.reshape(NBAND, 8, NITEM)
    relayout = pl.kernel(
        _relayout_body,
        out_type=jax.ShapeDtypeStruct((NITEM * RANK,), jnp.float32),
        mesh=mesh,
        compiler_params=pltpu.CompilerParams(needs_layout_passes=False),
        scratch_types=[
            pltpu.VMEM((2, NBAND, 8, GROUP), jnp.float32),
            pltpu.VMEM((2, GW), jnp.float32),
            pltpu.VMEM((NBAND, 8, TAIL_ROWS), jnp.float32),
            pltpu.VMEM((TAIL_ROWS * RANK,), jnp.float32),
            pltpu.SemaphoreType.DMA,
            pltpu.SemaphoreType.DMA,
            pltpu.SemaphoreType.DMA,
            pltpu.SemaphoreType.DMA,
        ],
    )
    item_rows = relayout(ei3).reshape(NITEM, RANK)

    wb = jnp.concatenate(
        [W.reshape(RANK), jnp.full((LANES,), b.reshape(-1)[0], jnp.float32)]
    ).astype(jnp.float32)

    run = pl.kernel(
        _gmf_body,
        out_type=jax.ShapeDtypeStruct((BATCH,), jnp.float32),
        mesh=mesh,
        compiler_params=pltpu.CompilerParams(
            needs_layout_passes=False, use_tc_tiling_on_sc=False),
        scratch_types=[
            pltpu.VMEM((NCHUNK, IDX_CHUNK), jnp.int32),
            pltpu.VMEM((NCHUNK, IDX_CHUNK), jnp.int32),
            pltpu.VMEM((BPW, RANK), jnp.float32),
            pltpu.VMEM((BPW, RANK), jnp.float32),
            pltpu.VMEM((LANES * BPW,), jnp.float32),
            pltpu.VMEM((BPW,), jnp.float32),
            pltpu.VMEM((RANK + LANES,), jnp.float32),
            pltpu.SemaphoreType.DMA,
        ],
    )
    return run(user.astype(jnp.int32), item.astype(jnp.int32),
               embed_user, item_rows, wb)


# speedup vs baseline: 1.0277x; 1.0122x over previous
import jax
import jax.numpy as jnp
from jax import lax
from jax.experimental import pallas as pl
from jax.experimental.pallas import tpu as pltpu
from jax.experimental.pallas import tpu_sc as plsc

NUM_CORES = 2
NUM_SUBCORES = 16
NUM_WORKERS = NUM_CORES * NUM_SUBCORES
LANES = 16
BATCH = 16384
RANK = 32
NBAND = RANK // 8
NITEM = 1000001
GROUP = 512
NGROups_FULL = NITEM // GROUP
TAIL_ROWS = NITEM - NGROups_FULL * GROUP
GW = 16384
BPW = BATCH // NUM_WORKERS
IDX_CHUNK = 128
NCHUNK = BPW // IDX_CHUNK


def _relayout_body(ei3_h, rows_h, inb_v, outb_v, tail_v, tout_v,
                   sem_i0, sem_i1, sem_o0, sem_o1):
    wid = lax.axis_index("s") * NUM_CORES + lax.axis_index("c")
    trips = jnp.where(wid == 0, 62, 61)
    start_g = 61 * wid + jnp.minimum(wid, 1)

    def fire_in(g, par, sem):
        row0 = (start_g + g) * GROUP
        for k in range(NBAND):
            for j in range(8):
                c = k * 8 + j
                pltpu.async_copy(
                    ei3_h.at[k, j, pl.ds(row0, GROUP)],
                    inb_v.at[pl.ds((par * RANK + c) * GROUP, GROUP)], sem)

    fire_in(0, 0, sem_i0)
    fire_in(1, 1, sem_i1)
    pltpu.async_copy(rows_h.at[pl.ds(0, GW)], outb_v.at[0], sem_o0)
    pltpu.async_copy(rows_h.at[pl.ds(0, GW)], outb_v.at[1], sem_o1)

    ci = lax.broadcasted_iota(jnp.int32, (LANES,), 0)
    k_lo = ci // 8
    j_vec = ci % 8
    k_hi = k_lo + 2
    pat_lo = ci * GROUP
    pat_hi = (ci + LANES) * GROUP

    def drain_in(par):
        pltpu.make_async_copy(
            ei3_h.at[0, 0, pl.ds(0, GW)],
            inb_v.at[pl.ds(par * RANK * GROUP, GW)],
            sem_i0 if par == 0 else sem_i1).wait()

    def drain_out(par):
        pltpu.make_async_copy(rows_h.at[pl.ds(0, GW)], outb_v.at[par],
                              sem_o0 if par == 0 else sem_o1).wait()

    UNROLL = 8

    def group_iter(g, carry):
        par = g % 2
        row0 = (start_g + g) * GROUP
        nxt = jnp.minimum(g + 2, trips - 1)

        @pl.when(par == 0)
        def _():
            drain_out(0)
            drain_in(0)

        @pl.when(par == 1)
        def _():
            drain_out(1)
            drain_in(1)

        po = jnp.full((LANES,), par * (RANK * GROUP), jnp.int32)
        lo_base = pat_lo + po
        hi_base = pat_hi + po

        def rows_iter(xb, c2):
            for u in range(UNROLL):
                x = xb * UNROLL + u
                xv = jnp.full((LANES,), x, jnp.int32)
                lo = plsc.load_gather(inb_v, [lo_base + xv])
                hi = plsc.load_gather(inb_v, [hi_base + xv])
                outb_v[par, pl.ds(x * RANK, LANES)] = lo
                outb_v[par, pl.ds(x * RANK + LANES, LANES)] = hi
            return c2

        lax.fori_loop(0, GROUP // UNROLL, rows_iter, 0)

        @pl.when(par == 0)
        def _():
            pltpu.async_copy(outb_v.at[0],
                             rows_h.at[pl.ds(row0 * RANK, GW)], sem_o0)
            fire_in(nxt, 0, sem_i0)

        @pl.when(par == 1)
        def _():
            pltpu.async_copy(outb_v.at[1],
                             rows_h.at[pl.ds(row0 * RANK, GW)], sem_o1)
            fire_in(nxt, 1, sem_i1)

        return carry

    lax.fori_loop(0, trips, group_iter, 0)

    drain_in(0)
    drain_in(1)
    drain_out(0)
    drain_out(1)

    @pl.when(wid == NUM_WORKERS - 1)
    def _():
        t0 = NGROups_FULL * GROUP
        pltpu.sync_copy(ei3_h.at[:, :, pl.ds(t0, TAIL_ROWS)], tail_v)
        for x in range(TAIL_ROWS):
            xv = jnp.full((LANES,), x, jnp.int32)
            lo = plsc.load_gather(tail_v, [k_lo, j_vec, xv])
            hi = plsc.load_gather(tail_v, [k_hi, j_vec, xv])
            tout_v[pl.ds(x * RANK, LANES)] = lo
            tout_v[pl.ds(x * RANK + LANES, LANES)] = hi
        pltpu.sync_copy(tout_v, rows_h.at[pl.ds(t0 * RANK, TAIL_ROWS * RANK)])


def _gmf_body(user_h, item_h, eu_h, ei_h, wb_h, out_h,
              idxu_v, idxi_v, ru_v, ri_v, pt_v, out_v, wb_v, sem):
    wid = lax.axis_index("s") * NUM_CORES + lax.axis_index("c")
    base = wid * BPW

    pltpu.sync_copy(wb_h, wb_v)
    for c in range(NCHUNK):
        pltpu.sync_copy(user_h.at[pl.ds(base + c * IDX_CHUNK, IDX_CHUNK)],
                        idxu_v.at[c])
        pltpu.sync_copy(item_h.at[pl.ds(base + c * IDX_CHUNK, IDX_CHUNK)],
                        idxi_v.at[c])

    copies = []
    for c in range(NCHUNK):
        copies.append(pltpu.async_copy(
            eu_h.at[idxu_v.at[c]], ru_v.at[pl.ds(c * IDX_CHUNK, IDX_CHUNK)], sem))
        copies.append(pltpu.async_copy(
            ei_h.at[idxi_v.at[c]], ri_v.at[pl.ds(c * IDX_CHUNK, IDX_CHUNK)], sem))
    for cp in copies:
        cp.wait()

    w0 = wb_v[pl.ds(0, LANES)]
    w1 = wb_v[pl.ds(LANES, LANES)]
    bv = wb_v[pl.ds(2 * LANES, LANES)]
    lane_iota = lax.broadcasted_iota(jnp.int32, (LANES,), 0)

    UNROLL = 8

    def stage_a(ib, carry):
        for k in range(UNROLL):
            i = ib * UNROLL + k
            p = (ru_v[i, pl.ds(0, LANES)] * ri_v[i, pl.ds(0, LANES)] * w0
                 + ru_v[i, pl.ds(LANES, LANES)] * ri_v[i, pl.ds(LANES, LANES)] * w1)
            plsc.store_scatter(pt_v, [lane_iota * BPW + i], p)
        return carry

    lax.fori_loop(0, BPW // UNROLL, stage_a, 0)

    def stage_b(bi, carry):
        acc = bv
        for l in range(LANES):
            acc = acc + pt_v[pl.ds(l * BPW + bi * LANES, LANES)]
        out_v[pl.ds(bi * LANES, LANES)] = acc
        return carry

    lax.fori_loop(0, BPW // LANES, stage_b, 0)

    pltpu.sync_copy(out_v, out_h.at[pl.ds(base, BPW)])


def kernel(user, item, embed_user, embed_item, W, b):
    mesh = plsc.VectorSubcoreMesh(core_axis_name="c", subcore_axis_name="s")

    ei3 = embed_item.T.reshape(NBAND, 8, NITEM)
    relayout = pl.kernel(
        _relayout_body,
        out_type=jax.ShapeDtypeStruct((NITEM * RANK,), jnp.float32),
        mesh=mesh,
        compiler_params=pltpu.CompilerParams(needs_layout_passes=False),
        scratch_types=[
            pltpu.VMEM((2 * RANK * GROUP,), jnp.float32),
            pltpu.VMEM((2, GW), jnp.float32),
            pltpu.VMEM((NBAND, 8, TAIL_ROWS), jnp.float32),
            pltpu.VMEM((TAIL_ROWS * RANK,), jnp.float32),
            pltpu.SemaphoreType.DMA,
            pltpu.SemaphoreType.DMA,
            pltpu.SemaphoreType.DMA,
            pltpu.SemaphoreType.DMA,
        ],
    )
    item_rows = relayout(ei3).reshape(NITEM, RANK)

    wb = jnp.concatenate(
        [W.reshape(RANK), jnp.full((LANES,), b.reshape(-1)[0], jnp.float32)]
    ).astype(jnp.float32)

    run = pl.kernel(
        _gmf_body,
        out_type=jax.ShapeDtypeStruct((BATCH,), jnp.float32),
        mesh=mesh,
        compiler_params=pltpu.CompilerParams(
            needs_layout_passes=False, use_tc_tiling_on_sc=False),
        scratch_types=[
            pltpu.VMEM((NCHUNK, IDX_CHUNK), jnp.int32),
            pltpu.VMEM((NCHUNK, IDX_CHUNK), jnp.int32),
            pltpu.VMEM((BPW, RANK), jnp.float32),
            pltpu.VMEM((BPW, RANK), jnp.float32),
            pltpu.VMEM((LANES * BPW,), jnp.float32),
            pltpu.VMEM((BPW,), jnp.float32),
            pltpu.VMEM((RANK + LANES,), jnp.float32),
            pltpu.SemaphoreType.DMA,
        ],
    )
    return run(user.astype(jnp.int32), item.astype(jnp.int32),
               embed_user, item_rows, wb)
